# baseline (speedup 1.0000x reference)
import functools

import jax
import jax.numpy as jnp
from jax import lax
from jax.experimental import pallas as pl
from jax.experimental.pallas import tpu as pltpu
from jax.experimental.pallas import tpu_sc as plsc

E = 160000
T = 640000
EMB = 128
INT = 64

_F32 = jnp.float32

_BE = 3200


def _edge_body(m_ref, rbfT_ref, wrbf_ref, wkj_ref, bkj_ref, wdown_ref,
               xkj_ref):
    mb = m_ref[...]
    t = jnp.dot(mb, wkj_ref[...], preferred_element_type=_F32) + bkj_ref[...]
    rbf_e = lax.dot_general(rbfT_ref[...], wrbf_ref[...],
                            dimension_numbers=(((0,), (0,)), ((), ())),
                            preferred_element_type=_F32)
    xkj_ref[...] = jnp.dot(t * rbf_e, wdown_ref[...], preferred_element_type=_F32)


def _edge_transform(m, rbfT, wrbf, wkj, bkj, wdown):
    nb = E // _BE
    row = lambda i: (i, 0)
    rep = lambda i: (0, 0)
    return pl.pallas_call(
        _edge_body,
        grid=(nb,),
        in_specs=[
            pl.BlockSpec((_BE, EMB), row),
            pl.BlockSpec((rbfT.shape[0], _BE), lambda i: (0, i)),
            pl.BlockSpec(wrbf.shape, rep),
            pl.BlockSpec(wkj.shape, rep),
            pl.BlockSpec(bkj.shape, rep),
            pl.BlockSpec(wdown.shape, rep),
        ],
        out_specs=pl.BlockSpec((_BE, INT), row),
        out_shape=jax.ShapeDtypeStruct((E, INT), _F32),
    )(m, rbfT, wrbf, wkj, bkj, wdown)


_BT = 5120


def _sbf_body(sbfT_ref, wsbf_ref, out_ref):
    out_ref[...] = lax.dot_general(sbfT_ref[...], wsbf_ref[...],
                                   dimension_numbers=(((0,), (0,)), ((), ())),
                                   preferred_element_type=_F32)


def _sbf_transform(sbfT, wsbf):
    nb = T // _BT
    return pl.pallas_call(
        _sbf_body,
        grid=(nb,),
        in_specs=[
            pl.BlockSpec((sbfT.shape[0], _BT), lambda i: (0, i)),
            pl.BlockSpec(wsbf.shape, lambda i: (0, 0)),
        ],
        out_specs=pl.BlockSpec((_BT, INT), lambda i: (i, 0)),
        out_shape=jax.ShapeDtypeStruct((T, INT), _F32),
    )(sbfT, wsbf)


_NC = 2
_NS = 16
_NP = 10
_PPC = _NP // _NC
_R = E // _NP
_RT = _R // _NS
_ZB = 40
_SEG = 2000
_TPT = T // _NS
_NSEG = _TPT // _SEG
_C2 = 64
_SH = 6
_NCH = (_SEG + _C2 + 16 + _C2 - 1) // _C2
_NB = 6
_PD = _NB - 2


def _segment_messages(xkj, sbft, src, dst):
    mesh = plsc.VectorSubcoreMesh(core_axis_name="c", subcore_axis_name="s",
                                  num_cores=_NC, num_subcores=_NS)

    @functools.partial(
        pl.kernel,
        out_type=jax.ShapeDtypeStruct((E, INT), _F32),
        mesh=mesh,
        compiler_params=pltpu.CompilerParams(needs_layout_passes=False,
                                             use_tc_tiling_on_sc=False),
        scratch_types=[
            pltpu.VMEM((_SEG,), jnp.int32),
            pltpu.VMEM((_SEG,), jnp.int32),
            pltpu.VMEM((_NCH, _C2), jnp.int32),
            pltpu.VMEM((_NCH, _C2), jnp.int32),
            pltpu.VMEM((_NCH, _C2), jnp.int32),
            pltpu.VMEM((_NB, _C2, INT), _F32),
            pltpu.VMEM((_NB, _C2, INT), _F32),
            pltpu.VMEM((_ZB, INT), _F32),
            pltpu.VMEM_SHARED((_R + 16, INT), _F32),
        ] + [pltpu.SemaphoreType.DMA] * (2 * _NB + 1),
    )
    def k(xkj_hbm, sbft_hbm, src_hbm, dst_hbm, out_hbm,
          dstb, srcb, csrc, ctid, cdst, gxb, gsb, zb, acc, *sems):
        semc = sems[2 * _NB]
        c = lax.axis_index("c")
        s = lax.axis_index("s")
        iota = lax.broadcasted_iota(jnp.int32, (16,), 0)

        def zrow(r, carry):
            for kk in range(INT // 16):
                zb[r, pl.ds(kk * 16, 16)] = jnp.zeros((16,), _F32)
            return carry

        lax.fori_loop(0, _ZB, zrow, 0)

        def do_pass(p, carry):
            base = (c * _PPC + p) * _R

            def zcp(z, cz):
                pltpu.sync_copy(zb, acc.at[pl.ds(s * _RT + z * _ZB, _ZB)])
                return cz

            lax.fori_loop(0, _RT // _ZB, zcp, 0)
            plsc.subcore_barrier()

            def do_seg(g, cg):
                t0 = s * _TPT + g * _SEG
                with jax.named_scope("seg_scan"):
                    pltpu.sync_copy(dst_hbm.at[pl.ds(t0, _SEG)], dstb)
                    pltpu.sync_copy(src_hbm.at[pl.ds(t0, _SEG)], srcb)

                    @plsc.parallel_loop(0, _SEG // 16, unroll=4,
                                        carry=jnp.zeros((16,), jnp.int32))
                    def scan(v, ptr):
                        off = v * 16
                        d = dstb[pl.ds(off, 16)]
                        sv = srcb[pl.ds(off, 16)]
                        dl = d - base
                        msk = (dl >= 0) & (dl < _R)
                        incl = plsc.cumsum(msk.astype(jnp.int32))
                        pos = ptr + incl - 1
                        prow = lax.shift_right_logical(pos, _SH)
                        pcol = lax.bitwise_and(pos, _C2 - 1)
                        tid = t0 + off + iota
                        plsc.store_scatter(csrc, [prow, pcol], sv, mask=msk)
                        plsc.store_scatter(ctid, [prow, pcol], tid, mask=msk)
                        plsc.store_scatter(cdst, [prow, pcol], dl, mask=msk)
                        return ptr + plsc.all_reduce_population_count(msk)

                ptr = scan
                ncomp = jnp.max(ptr)
                ncv = jnp.full((16,), 0, jnp.int32) + ncomp
                zero16 = jnp.zeros((16,), jnp.int32)
                dump16 = jnp.full((16,), _R, jnp.int32)
                for kk in range(_C2 // 16):
                    ppos = ncv + (kk * 16) + iota
                    prow = lax.shift_right_logical(ppos, _SH)
                    pcol = lax.bitwise_and(ppos, _C2 - 1)
                    plsc.store_scatter(csrc, [prow, pcol], zero16)
                    plsc.store_scatter(ctid, [prow, pcol], zero16)
                    plsc.store_scatter(cdst, [prow, pcol], dump16)
                nch = (ncomp + _C2 - 1) // _C2

                def _gather(c, b):
                    pltpu.async_copy(xkj_hbm.at[csrc.at[c]], gxb.at[b],
                                     sems[2 * b])
                    pltpu.async_copy(sbft_hbm.at[ctid.at[c]], gsb.at[b],
                                     sems[2 * b + 1])

                def _drain_one_scatter():
                    pltpu.make_async_copy(gxb.at[0], acc.at[cdst.at[0]],
                                          semc).wait()

                for c in range(_PD):
                    @pl.when(c < nch)
                    def _prime():
                        _gather(c, c)

                def drain_quint(j5, cj):
                    for b in range(_NB):
                        j = j5 * _NB + b

                        @pl.when(j < nch)
                        def _chunk():
                            @pl.when(j >= 2)
                            def _dr():
                                _drain_one_scatter()

                            @pl.when(j + _PD < nch)
                            def _prefetch():
                                _gather(j + _PD, (b + _PD) % _NB)

                            pltpu.make_async_copy(
                                xkj_hbm.at[csrc.at[j]], gxb.at[b],
                                sems[2 * b]).wait()
                            pltpu.make_async_copy(
                                sbft_hbm.at[ctid.at[j]], gsb.at[b],
                                sems[2 * b + 1]).wait()

                            @plsc.parallel_loop(0, _C2, unroll=8)
                            def _mrow(r):
                                for kk in range(INT // 16):
                                    sl = pl.ds(kk * 16, 16)
                                    gxb[b, r, sl] = gxb[b, r, sl] * gsb[b, r, sl]

                            pltpu.async_copy(gxb.at[b], acc.at[cdst.at[j]],
                                             semc, add=True)
                    return cj

                lax.fori_loop(0, (nch + _NB - 1) // _NB, drain_quint, 0)

                @pl.when(nch >= 1)
                def _final_drain1():
                    _drain_one_scatter()

                @pl.when(nch >= 2)
                def _final_drain2():
                    _drain_one_scatter()
                return cg

            lax.fori_loop(0, _NSEG, do_seg, 0)
            plsc.subcore_barrier()
            pltpu.sync_copy(acc.at[pl.ds(s * _RT, _RT)],
                            out_hbm.at[pl.ds(base + s * _RT, _RT)])
            return carry

        lax.fori_loop(0, _PPC, do_pass, 0)

    return k(xkj, sbft, src, dst)


def _tail_body(macc_ref, m_ref, wji, bji, wup, wb01, bb01, wb02, bb02, wf, bf,
               wa01, ba01, wa02, ba02, wa11, ba11, wa12, ba12, out_ref):
    dot = lambda a, b: jnp.dot(a, b[...], preferred_element_type=_F32)
    x_ji = dot(m_ref[...], wji) + bji[...]
    mu = dot(macc_ref[...], wup) + x_ji
    mu = mu + dot(dot(mu, wb01) + bb01[...], wb02) + bb02[...]
    mu = dot(mu, wf) + bf[...]
    mn = m_ref[...] + mu
    mn = mn + dot(dot(mn, wa01) + ba01[...], wa02) + ba02[...]
    mn = mn + dot(dot(mn, wa11) + ba11[...], wa12) + ba12[...]
    out_ref[...] = mn


def _tail(macc, m, *ws):
    nb = E // _BE
    row = lambda i: (i, 0)
    rep = lambda i: (0, 0)
    return pl.pallas_call(
        _tail_body,
        grid=(nb,),
        in_specs=[
            pl.BlockSpec((_BE, INT), row),
            pl.BlockSpec((_BE, EMB), row),
        ] + [pl.BlockSpec(w.shape, rep) for w in ws],
        out_specs=pl.BlockSpec((_BE, EMB), row),
        out_shape=jax.ShapeDtypeStruct((E, EMB), _F32),
    )(macc, m, *ws)


def kernel(m, rbf, sbf, lg_edge_index, W_rbf1, W_rbf2, W_sbf1, W_sbf2, W_ji,
           b_ji, W_kj, b_kj, W_down, W_up, Wb0_1, bb0_1, Wb0_2, bb0_2,
           W_final, b_final, Wa0_1, ba0_1, Wa0_2, ba0_2, Wa1_1, ba1_1, Wa1_2,
           ba1_2):
    wrbf = jnp.dot(W_rbf1, W_rbf2, preferred_element_type=_F32)
    wsbf = jnp.dot(W_sbf1, W_sbf2, preferred_element_type=_F32)
    r1 = lambda b: b.reshape(1, -1)

    xkj = _edge_transform(m, rbf.T, wrbf, W_kj, r1(b_kj), W_down)
    sbft = _sbf_transform(sbf.T, wsbf)
    src = lg_edge_index[0]
    dst = lg_edge_index[1]
    macc = _segment_messages(xkj, sbft, src, dst)
    return _tail(macc, m, W_ji, r1(b_ji), W_up, Wb0_1, r1(bb0_1), Wb0_2,
                 r1(bb0_2), W_final, r1(b_final), Wa0_1, r1(ba0_1), Wa0_2,
                 r1(ba0_2), Wa1_1, r1(ba1_1), Wa1_2, r1(ba1_2))


# speedup vs baseline: 1.5347x; 1.5347x over previous
import functools

import jax
import jax.numpy as jnp
from jax import lax
from jax.experimental import pallas as pl
from jax.experimental.pallas import tpu as pltpu
from jax.experimental.pallas import tpu_sc as plsc

E = 160000
T = 640000
EMB = 128
INT = 64

_F32 = jnp.float32

_BE = 3200


def _edge_body(m_ref, rbf_ref, wrbf_ref, wkj_ref, bkj_ref, wdown_ref,
               xkj_ref):
    mb = m_ref[...]
    t = jnp.dot(mb, wkj_ref[...], preferred_element_type=_F32) + bkj_ref[...]
    rbf_e = jnp.dot(rbf_ref[...], wrbf_ref[...], preferred_element_type=_F32)
    xkj_ref[...] = jnp.dot(t * rbf_e, wdown_ref[...], preferred_element_type=_F32)


def _edge_transform(m, rbf, wrbf, wkj, bkj, wdown):
    nb = E // _BE
    row = lambda i: (i, 0)
    rep = lambda i: (0, 0)
    return pl.pallas_call(
        _edge_body,
        grid=(nb,),
        in_specs=[
            pl.BlockSpec((_BE, EMB), row),
            pl.BlockSpec((_BE, rbf.shape[1]), row),
            pl.BlockSpec(wrbf.shape, rep),
            pl.BlockSpec(wkj.shape, rep),
            pl.BlockSpec(bkj.shape, rep),
            pl.BlockSpec(wdown.shape, rep),
        ],
        out_specs=pl.BlockSpec((_BE, INT), row),
        out_shape=jax.ShapeDtypeStruct((E, INT), _F32),
    )(m, rbf, wrbf, wkj, bkj, wdown)


_BT = 5120


def _sbf_body(sbfT_ref, wsbf_ref, out_ref):
    out_ref[...] = lax.dot_general(sbfT_ref[...], wsbf_ref[...],
                                   dimension_numbers=(((0,), (0,)), ((), ())),
                                   preferred_element_type=_F32)


def _sbf_transform(sbfT, wsbf):
    nb = T // _BT
    return pl.pallas_call(
        _sbf_body,
        grid=(nb,),
        in_specs=[
            pl.BlockSpec((sbfT.shape[0], _BT), lambda i: (0, i)),
            pl.BlockSpec(wsbf.shape, lambda i: (0, 0)),
        ],
        out_specs=pl.BlockSpec((_BT, INT), lambda i: (i, 0)),
        out_shape=jax.ShapeDtypeStruct((T, INT), _F32),
    )(sbfT, wsbf)


_NC = 2
_NS = 16
_NP = 10
_PPC = _NP // _NC
_R = E // _NP
_RT = _R // _NS
_ZB = 40
_SEG = 4000
_TPT = T // _NS
_NSEG = _TPT // _SEG
_C2 = 64
_SH = 6
_NCH = (_SEG + _C2 + 16 + _C2 - 1) // _C2
_NB = 5
_PD = _NB - 2


def _segment_messages(xkj, sbft, src, dst):
    mesh = plsc.VectorSubcoreMesh(core_axis_name="c", subcore_axis_name="s",
                                  num_cores=_NC, num_subcores=_NS)

    @functools.partial(
        pl.kernel,
        out_type=jax.ShapeDtypeStruct((E, INT), _F32),
        mesh=mesh,
        compiler_params=pltpu.CompilerParams(needs_layout_passes=False,
                                             use_tc_tiling_on_sc=False),
        scratch_types=[
            pltpu.VMEM((_SEG,), jnp.int32),
            pltpu.VMEM((_SEG,), jnp.int32),
            pltpu.VMEM((_NCH, _C2), jnp.int32),
            pltpu.VMEM((_NCH, _C2), jnp.int32),
            pltpu.VMEM((_NCH, _C2), jnp.int32),
            pltpu.VMEM((_NB, _C2, INT), _F32),
            pltpu.VMEM((_NB, _C2, INT), _F32),
            pltpu.VMEM((_ZB, INT), _F32),
            pltpu.VMEM_SHARED((_R + 16, INT), _F32),
        ] + [pltpu.SemaphoreType.DMA] * (2 * _NB + 1),
    )
    def k(xkj_hbm, sbft_hbm, src_hbm, dst_hbm, out_hbm,
          dstb, srcb, csrc, ctid, cdst, gxb, gsb, zb, acc, *sems):
        semc = sems[2 * _NB]
        c = lax.axis_index("c")
        s = lax.axis_index("s")
        iota = lax.broadcasted_iota(jnp.int32, (16,), 0)

        def zrow(r, carry):
            for kk in range(INT // 16):
                zb[r, pl.ds(kk * 16, 16)] = jnp.zeros((16,), _F32)
            return carry

        lax.fori_loop(0, _ZB, zrow, 0)

        def do_pass(p, carry):
            base = (c * _PPC + p) * _R

            def zcp(z, cz):
                pltpu.sync_copy(zb, acc.at[pl.ds(s * _RT + z * _ZB, _ZB)])
                return cz

            lax.fori_loop(0, _RT // _ZB, zcp, 0)
            plsc.subcore_barrier()

            def do_seg(g, cg):
                t0 = s * _TPT + g * _SEG
                with jax.named_scope("seg_scan"):
                    pltpu.sync_copy(dst_hbm.at[pl.ds(t0, _SEG)], dstb)
                    pltpu.sync_copy(src_hbm.at[pl.ds(t0, _SEG)], srcb)

                    @plsc.parallel_loop(0, _SEG // 16, unroll=2,
                                        carry=jnp.zeros((16,), jnp.int32))
                    def scan(v, ptr):
                        off = v * 16
                        d = dstb[pl.ds(off, 16)]
                        sv = srcb[pl.ds(off, 16)]
                        dl = d - base
                        msk = (dl >= 0) & (dl < _R)
                        incl = plsc.cumsum(msk.astype(jnp.int32))
                        pos = ptr + incl - 1
                        prow = lax.shift_right_logical(pos, _SH)
                        pcol = lax.bitwise_and(pos, _C2 - 1)
                        tid = t0 + off + iota
                        plsc.store_scatter(csrc, [prow, pcol], sv, mask=msk)
                        plsc.store_scatter(ctid, [prow, pcol], tid, mask=msk)
                        plsc.store_scatter(cdst, [prow, pcol], dl, mask=msk)
                        return ptr + plsc.all_reduce_population_count(msk)

                ptr = scan
                ncomp = jnp.max(ptr)
                ncv = jnp.full((16,), 0, jnp.int32) + ncomp
                zero16 = jnp.zeros((16,), jnp.int32)
                dump16 = jnp.full((16,), _R, jnp.int32)
                for kk in range(_C2 // 16):
                    ppos = ncv + (kk * 16) + iota
                    prow = lax.shift_right_logical(ppos, _SH)
                    pcol = lax.bitwise_and(ppos, _C2 - 1)
                    plsc.store_scatter(csrc, [prow, pcol], zero16)
                    plsc.store_scatter(ctid, [prow, pcol], zero16)
                    plsc.store_scatter(cdst, [prow, pcol], dump16)
                nch = (ncomp + _C2 - 1) // _C2

                def _gather(c, b):
                    pltpu.async_copy(xkj_hbm.at[csrc.at[c]], gxb.at[b],
                                     sems[2 * b])
                    pltpu.async_copy(sbft_hbm.at[ctid.at[c]], gsb.at[b],
                                     sems[2 * b + 1])

                def _drain_one_scatter():
                    pltpu.make_async_copy(gxb.at[0], acc.at[cdst.at[0]],
                                          semc).wait()

                for c in range(_PD):
                    @pl.when(c < nch)
                    def _prime():
                        _gather(c, c)

                def drain_quint(j5, cj):
                    for b in range(_NB):
                        j = j5 * _NB + b

                        @pl.when(j < nch)
                        def _chunk():
                            @pl.when(j >= 2)
                            def _dr():
                                _drain_one_scatter()

                            @pl.when(j + _PD < nch)
                            def _prefetch():
                                _gather(j + _PD, (b + _PD) % _NB)

                            pltpu.make_async_copy(
                                xkj_hbm.at[csrc.at[j]], gxb.at[b],
                                sems[2 * b]).wait()
                            pltpu.make_async_copy(
                                sbft_hbm.at[ctid.at[j]], gsb.at[b],
                                sems[2 * b + 1]).wait()

                            @plsc.parallel_loop(0, _C2, unroll=8)
                            def _mrow(r):
                                for kk in range(INT // 16):
                                    sl = pl.ds(kk * 16, 16)
                                    gxb[b, r, sl] = gxb[b, r, sl] * gsb[b, r, sl]

                            pltpu.async_copy(gxb.at[b], acc.at[cdst.at[j]],
                                             semc, add=True)
                    return cj

                lax.fori_loop(0, (nch + _NB - 1) // _NB, drain_quint, 0)

                @pl.when(nch >= 1)
                def _final_drain1():
                    _drain_one_scatter()

                @pl.when(nch >= 2)
                def _final_drain2():
                    _drain_one_scatter()
                return cg

            lax.fori_loop(0, _NSEG, do_seg, 0)
            plsc.subcore_barrier()
            pltpu.sync_copy(acc.at[pl.ds(s * _RT, _RT)],
                            out_hbm.at[pl.ds(base + s * _RT, _RT)])
            return carry

        lax.fori_loop(0, _PPC, do_pass, 0)

    return k(xkj, sbft, src, dst)


def _tail_body(macc_ref, m_ref, wji, bji, wup, wb01, bb01, wb02, bb02, wf, bf,
               wa01, ba01, wa02, ba02, wa11, ba11, wa12, ba12, out_ref):
    dot = lambda a, b: jnp.dot(a, b[...], preferred_element_type=_F32)
    x_ji = dot(m_ref[...], wji) + bji[...]
    mu = dot(macc_ref[...], wup) + x_ji
    mu = mu + dot(dot(mu, wb01) + bb01[...], wb02) + bb02[...]
    mu = dot(mu, wf) + bf[...]
    mn = m_ref[...] + mu
    mn = mn + dot(dot(mn, wa01) + ba01[...], wa02) + ba02[...]
    mn = mn + dot(dot(mn, wa11) + ba11[...], wa12) + ba12[...]
    out_ref[...] = mn


def _tail(macc, m, *ws):
    nb = E // _BE
    row = lambda i: (i, 0)
    rep = lambda i: (0, 0)
    return pl.pallas_call(
        _tail_body,
        grid=(nb,),
        in_specs=[
            pl.BlockSpec((_BE, INT), row),
            pl.BlockSpec((_BE, EMB), row),
        ] + [pl.BlockSpec(w.shape, rep) for w in ws],
        out_specs=pl.BlockSpec((_BE, EMB), row),
        out_shape=jax.ShapeDtypeStruct((E, EMB), _F32),
    )(macc, m, *ws)


def kernel(m, rbf, sbf, lg_edge_index, W_rbf1, W_rbf2, W_sbf1, W_sbf2, W_ji,
           b_ji, W_kj, b_kj, W_down, W_up, Wb0_1, bb0_1, Wb0_2, bb0_2,
           W_final, b_final, Wa0_1, ba0_1, Wa0_2, ba0_2, Wa1_1, ba1_1, Wa1_2,
           ba1_2):
    wrbf = jnp.dot(W_rbf1, W_rbf2, preferred_element_type=_F32)
    wsbf = jnp.dot(W_sbf1, W_sbf2, preferred_element_type=_F32)
    r1 = lambda b: b.reshape(1, -1)

    xkj = _edge_transform(m, rbf, wrbf, W_kj, r1(b_kj), W_down)
    sbft = _sbf_transform(sbf.T, wsbf)
    src = lg_edge_index[0]
    dst = lg_edge_index[1]
    macc = _segment_messages(xkj, sbft, src, dst)
    return _tail(macc, m, W_ji, r1(b_ji), W_up, Wb0_1, r1(bb0_1), Wb0_2,
                 r1(bb0_2), W_final, r1(b_final), Wa0_1, r1(ba0_1), Wa0_2,
                 r1(ba0_2), Wa1_1, r1(ba1_1), Wa1_2, r1(ba1_2))


# speedup vs baseline: 1.5729x; 1.0249x over previous
import functools

import jax
import jax.numpy as jnp
from jax import lax
from jax.experimental import pallas as pl
from jax.experimental.pallas import tpu as pltpu
from jax.experimental.pallas import tpu_sc as plsc

E = 160000
T = 640000
EMB = 128
INT = 64

_F32 = jnp.float32

_BE = 3200


def _edge_body(m_ref, rbfT_ref, wrbf_ref, wkj_ref, bkj_ref, wdown_ref,
               xkj_ref):
    mb = m_ref[...]
    t = jnp.dot(mb, wkj_ref[...], preferred_element_type=_F32) + bkj_ref[...]
    rbf_e = lax.dot_general(rbfT_ref[...], wrbf_ref[...],
                            dimension_numbers=(((0,), (0,)), ((), ())),
                            preferred_element_type=_F32)
    xkj_ref[...] = jnp.dot(t * rbf_e, wdown_ref[...], preferred_element_type=_F32)


def _edge_transform(m, rbfT, wrbf, wkj, bkj, wdown):
    nb = E // _BE
    row = lambda i: (i, 0)
    rep = lambda i: (0, 0)
    return pl.pallas_call(
        _edge_body,
        grid=(nb,),
        in_specs=[
            pl.BlockSpec((_BE, EMB), row),
            pl.BlockSpec((rbfT.shape[0], _BE), lambda i: (0, i)),
            pl.BlockSpec(wrbf.shape, rep),
            pl.BlockSpec(wkj.shape, rep),
            pl.BlockSpec(bkj.shape, rep),
            pl.BlockSpec(wdown.shape, rep),
        ],
        out_specs=pl.BlockSpec((_BE, INT), row),
        out_shape=jax.ShapeDtypeStruct((E, INT), _F32),
    )(m, rbfT, wrbf, wkj, bkj, wdown)


_BT = 5120


def _sbf_body(sbfT_ref, wsbf_ref, out_ref):
    out_ref[...] = lax.dot_general(sbfT_ref[...], wsbf_ref[...],
                                   dimension_numbers=(((0,), (0,)), ((), ())),
                                   preferred_element_type=_F32)


def _sbf_transform(sbfT, wsbf):
    nb = T // _BT
    return pl.pallas_call(
        _sbf_body,
        grid=(nb,),
        in_specs=[
            pl.BlockSpec((sbfT.shape[0], _BT), lambda i: (0, i)),
            pl.BlockSpec(wsbf.shape, lambda i: (0, 0)),
        ],
        out_specs=pl.BlockSpec((_BT, INT), lambda i: (i, 0)),
        out_shape=jax.ShapeDtypeStruct((T, INT), _F32),
    )(sbfT, wsbf)


_NC = 2
_NS = 16
_NP = 10
_PPC = _NP // _NC
_R = E // _NP
_RT = _R // _NS
_ZB = 40
_SEG = 4000
_TPT = T // _NS
_NSEG = _TPT // _SEG
_C2 = 64
_SH = 6
_NCH = (_SEG + _C2 + 16 + _C2 - 1) // _C2
_NB = 5
_PD = _NB - 2


def _segment_messages(xkj, sbft, src, dst):
    mesh = plsc.VectorSubcoreMesh(core_axis_name="c", subcore_axis_name="s",
                                  num_cores=_NC, num_subcores=_NS)

    @functools.partial(
        pl.kernel,
        out_type=jax.ShapeDtypeStruct((E, INT), _F32),
        mesh=mesh,
        compiler_params=pltpu.CompilerParams(needs_layout_passes=False,
                                             use_tc_tiling_on_sc=False),
        scratch_types=[
            pltpu.VMEM((_SEG,), jnp.int32),
            pltpu.VMEM((_SEG,), jnp.int32),
            pltpu.VMEM((_NCH, _C2), jnp.int32),
            pltpu.VMEM((_NCH, _C2), jnp.int32),
            pltpu.VMEM((_NCH, _C2), jnp.int32),
            pltpu.VMEM((_NB, _C2, INT), _F32),
            pltpu.VMEM((_NB, _C2, INT), _F32),
            pltpu.VMEM((_ZB, INT), _F32),
            pltpu.VMEM_SHARED((_R + 16, INT), _F32),
        ] + [pltpu.SemaphoreType.DMA] * (2 * _NB + 1),
    )
    def k(xkj_hbm, sbft_hbm, src_hbm, dst_hbm, out_hbm,
          dstb, srcb, csrc, ctid, cdst, gxb, gsb, zb, acc, *sems):
        semc = sems[2 * _NB]
        c = lax.axis_index("c")
        s = lax.axis_index("s")
        iota = lax.broadcasted_iota(jnp.int32, (16,), 0)

        def zrow(r, carry):
            for kk in range(INT // 16):
                zb[r, pl.ds(kk * 16, 16)] = jnp.zeros((16,), _F32)
            return carry

        lax.fori_loop(0, _ZB, zrow, 0)

        def do_pass(p, carry):
            base = (c * _PPC + p) * _R

            def zcp(z, cz):
                pltpu.sync_copy(zb, acc.at[pl.ds(s * _RT + z * _ZB, _ZB)])
                return cz

            lax.fori_loop(0, _RT // _ZB, zcp, 0)
            plsc.subcore_barrier()

            def do_seg(g, cg):
                t0 = s * _TPT + g * _SEG
                with jax.named_scope("seg_scan"):
                    pltpu.sync_copy(dst_hbm.at[pl.ds(t0, _SEG)], dstb)
                    pltpu.sync_copy(src_hbm.at[pl.ds(t0, _SEG)], srcb)

                    @plsc.parallel_loop(0, _SEG // 16, unroll=4,
                                        carry=jnp.zeros((16,), jnp.int32))
                    def scan(v, ptr):
                        off = v * 16
                        d = dstb[pl.ds(off, 16)]
                        sv = srcb[pl.ds(off, 16)]
                        dl = d - base
                        msk = (dl >= 0) & (dl < _R)
                        incl = plsc.cumsum(msk.astype(jnp.int32))
                        pos = ptr + incl - 1
                        prow = lax.shift_right_logical(pos, _SH)
                        pcol = lax.bitwise_and(pos, _C2 - 1)
                        tid = t0 + off + iota
                        plsc.store_scatter(csrc, [prow, pcol], sv, mask=msk)
                        plsc.store_scatter(ctid, [prow, pcol], tid, mask=msk)
                        plsc.store_scatter(cdst, [prow, pcol], dl, mask=msk)
                        return ptr + plsc.all_reduce_population_count(msk)

                ptr = scan
                ncomp = jnp.max(ptr)
                ncv = jnp.full((16,), 0, jnp.int32) + ncomp
                zero16 = jnp.zeros((16,), jnp.int32)
                dump16 = jnp.full((16,), _R, jnp.int32)
                for kk in range(_C2 // 16):
                    ppos = ncv + (kk * 16) + iota
                    prow = lax.shift_right_logical(ppos, _SH)
                    pcol = lax.bitwise_and(ppos, _C2 - 1)
                    plsc.store_scatter(csrc, [prow, pcol], zero16)
                    plsc.store_scatter(ctid, [prow, pcol], zero16)
                    plsc.store_scatter(cdst, [prow, pcol], dump16)
                nch = (ncomp + _C2 - 1) // _C2

                def _gather(c, b):
                    pltpu.async_copy(xkj_hbm.at[csrc.at[c]], gxb.at[b],
                                     sems[2 * b])
                    pltpu.async_copy(sbft_hbm.at[ctid.at[c]], gsb.at[b],
                                     sems[2 * b + 1])

                def _drain_one_scatter():
                    pltpu.make_async_copy(gxb.at[0], acc.at[cdst.at[0]],
                                          semc).wait()

                for c in range(_PD):
                    @pl.when(c < nch)
                    def _prime():
                        _gather(c, c)

                def drain_quint(j5, cj):
                    for b in range(_NB):
                        j = j5 * _NB + b

                        @pl.when(j < nch)
                        def _chunk():
                            @pl.when(j >= 2)
                            def _dr():
                                _drain_one_scatter()

                            @pl.when(j + _PD < nch)
                            def _prefetch():
                                _gather(j + _PD, (b + _PD) % _NB)

                            pltpu.make_async_copy(
                                xkj_hbm.at[csrc.at[j]], gxb.at[b],
                                sems[2 * b]).wait()
                            pltpu.make_async_copy(
                                sbft_hbm.at[ctid.at[j]], gsb.at[b],
                                sems[2 * b + 1]).wait()

                            @plsc.parallel_loop(0, _C2, unroll=8)
                            def _mrow(r):
                                for kk in range(INT // 16):
                                    sl = pl.ds(kk * 16, 16)
                                    gxb[b, r, sl] = gxb[b, r, sl] * gsb[b, r, sl]

                            pltpu.async_copy(gxb.at[b], acc.at[cdst.at[j]],
                                             semc, add=True)
                    return cj

                lax.fori_loop(0, (nch + _NB - 1) // _NB, drain_quint, 0)

                @pl.when(nch >= 1)
                def _final_drain1():
                    _drain_one_scatter()

                @pl.when(nch >= 2)
                def _final_drain2():
                    _drain_one_scatter()
                return cg

            lax.fori_loop(0, _NSEG, do_seg, 0)
            plsc.subcore_barrier()
            pltpu.sync_copy(acc.at[pl.ds(s * _RT, _RT)],
                            out_hbm.at[pl.ds(base + s * _RT, _RT)])
            return carry

        lax.fori_loop(0, _PPC, do_pass, 0)

    return k(xkj, sbft, src, dst)


def _tail_body(macc_ref, m_ref, wji, bji, wup, wb01, bb01, wb02, bb02, wf, bf,
               wa01, ba01, wa02, ba02, wa11, ba11, wa12, ba12, out_ref):
    dot = lambda a, b: jnp.dot(a, b[...], preferred_element_type=_F32)
    x_ji = dot(m_ref[...], wji) + bji[...]
    mu = dot(macc_ref[...], wup) + x_ji
    mu = mu + dot(dot(mu, wb01) + bb01[...], wb02) + bb02[...]
    mu = dot(mu, wf) + bf[...]
    mn = m_ref[...] + mu
    mn = mn + dot(dot(mn, wa01) + ba01[...], wa02) + ba02[...]
    mn = mn + dot(dot(mn, wa11) + ba11[...], wa12) + ba12[...]
    out_ref[...] = mn


def _tail(macc, m, *ws):
    nb = E // _BE
    row = lambda i: (i, 0)
    rep = lambda i: (0, 0)
    return pl.pallas_call(
        _tail_body,
        grid=(nb,),
        in_specs=[
            pl.BlockSpec((_BE, INT), row),
            pl.BlockSpec((_BE, EMB), row),
        ] + [pl.BlockSpec(w.shape, rep) for w in ws],
        out_specs=pl.BlockSpec((_BE, EMB), row),
        out_shape=jax.ShapeDtypeStruct((E, EMB), _F32),
    )(macc, m, *ws)


def kernel(m, rbf, sbf, lg_edge_index, W_rbf1, W_rbf2, W_sbf1, W_sbf2, W_ji,
           b_ji, W_kj, b_kj, W_down, W_up, Wb0_1, bb0_1, Wb0_2, bb0_2,
           W_final, b_final, Wa0_1, ba0_1, Wa0_2, ba0_2, Wa1_1, ba1_1, Wa1_2,
           ba1_2):
    wrbf = jnp.dot(W_rbf1, W_rbf2, preferred_element_type=_F32)
    wsbf = jnp.dot(W_sbf1, W_sbf2, preferred_element_type=_F32)
    r1 = lambda b: b.reshape(1, -1)

    xkj = _edge_transform(m, rbf.T, wrbf, W_kj, r1(b_kj), W_down)
    sbft = _sbf_transform(sbf.T, wsbf)
    src = lg_edge_index[0]
    dst = lg_edge_index[1]
    macc = _segment_messages(xkj, sbft, src, dst)
    return _tail(macc, m, W_ji, r1(b_ji), W_up, Wb0_1, r1(bb0_1), Wb0_2,
                 r1(bb0_2), W_final, r1(b_final), Wa0_1, r1(ba0_1), Wa0_2,
                 r1(ba0_2), Wa1_1, r1(ba1_1), Wa1_2, r1(ba1_2))
